# baseline (speedup 1.0000x reference)
import jax
import jax.numpy as jnp
from jax import lax
from jax.experimental import pallas as pl
from jax.experimental.pallas import tpu as pltpu

N = 16384
D = 2048
E = 16
TILE = 2048
GRID = N // TILE
EPS = 1e-6


def _router_kernel(ua_ref, x_ref, w_ref, gates_ref, aux_ref, lbuf, imp_ref, load_ref):
    i = pl.program_id(0)

    @pl.when(i < GRID)
    def _():
        lbuf[i % 2] = lax.dot_general(
            w_ref[...], x_ref[...], (((1,), (1,)), ((), ())),
            preferred_element_type=jnp.float32,
        )

    @pl.when(i > 0)
    def _tail():
        logits = lbuf[(i - 1) % 2].T
        m = jnp.max(logits, axis=1, keepdims=True)
        e = jnp.exp(logits - m)
        s = jnp.sum(e, axis=1, keepdims=True)
        probs = e / s
        ids = lax.broadcasted_iota(jnp.int32, (TILE, E), 1)
        ismax = logits == m
        first = jnp.min(jnp.where(ismax, ids, E), axis=1, keepdims=True)
        gates = (ids == first).astype(jnp.float32)
        gates_ref[...] = gates
        imp_part = jnp.sum(probs, axis=0, keepdims=True)
        load_part = jnp.sum(gates, axis=0, keepdims=True)

        @pl.when(i == 1)
        def _():
            imp_ref[...] = imp_part
            load_ref[...] = load_part

        @pl.when(i > 1)
        def _():
            imp_ref[...] += imp_part
            load_ref[...] += load_part

        @pl.when(i == GRID)
        def _():
            imp = imp_ref[...]
            ld = load_ref[...]
            impn = imp / (jnp.sum(imp) + EPS)
            ldn = ld / (jnp.sum(ld) + EPS)
            d2 = (impn - ldn) ** 2
            aux_ref[...] = jnp.sum(d2, axis=1, keepdims=True) / E * ua_ref[0, 0]


def kernel(x, W, use_aux_loss):
    ua = jnp.asarray(use_aux_loss, jnp.float32).reshape(1, 1)
    gates, aux = pl.pallas_call(
        _router_kernel,
        grid=(GRID + 1,),
        in_specs=[
            pl.BlockSpec(memory_space=pltpu.SMEM),
            pl.BlockSpec((TILE, D), lambda i: (jnp.minimum(i, GRID - 1), 0)),
            pl.BlockSpec((E, D), lambda i: (0, 0)),
        ],
        out_specs=[
            pl.BlockSpec((TILE, E), lambda i: (jnp.maximum(i - 1, 0), 0)),
            pl.BlockSpec((1, 1), lambda i: (0, 0)),
        ],
        out_shape=[
            jax.ShapeDtypeStruct((N, E), jnp.float32),
            jax.ShapeDtypeStruct((1, 1), jnp.float32),
        ],
        scratch_shapes=[
            pltpu.VMEM((2, E, TILE), jnp.float32),
            pltpu.VMEM((1, E), jnp.float32),
            pltpu.VMEM((1, E), jnp.float32),
        ],
        compiler_params=pltpu.CompilerParams(
            dimension_semantics=("arbitrary",)
        ),
    )(ua, x, W)
    return gates, aux.reshape(())


# speedup vs baseline: 1.0510x; 1.0510x over previous
import jax
import jax.numpy as jnp
from jax import lax
from jax.experimental import pallas as pl
from jax.experimental.pallas import tpu as pltpu

N = 16384
D = 2048
E = 16
TILE = 2048
GRID = N // TILE
EPS = 1e-6


def _router_kernel(ua_ref, x_ref, w_ref, gates_ref, aux_ref, imp_ref, load_ref):
    i = pl.program_id(0)
    lt = lax.dot_general(
        w_ref[...], x_ref[...], (((1,), (1,)), ((), ())),
        preferred_element_type=jnp.float32,
    )
    logits = lt.T
    m = jnp.max(logits, axis=1, keepdims=True)
    e = jnp.exp(logits - m)
    s = jnp.sum(e, axis=1, keepdims=True)
    probs = e / s
    ids = lax.broadcasted_iota(jnp.int32, (TILE, E), 1)
    ismax = logits == m
    first = jnp.min(jnp.where(ismax, ids, E), axis=1, keepdims=True)
    gates = (ids == first).astype(jnp.float32)
    gates_ref[...] = gates
    imp_part = jnp.sum(probs, axis=0, keepdims=True)
    load_part = jnp.sum(gates, axis=0, keepdims=True)

    @pl.when(i == 0)
    def _():
        imp_ref[...] = imp_part
        load_ref[...] = load_part

    @pl.when(i > 0)
    def _():
        imp_ref[...] += imp_part
        load_ref[...] += load_part

    @pl.when(i == GRID - 1)
    def _():
        imp = imp_ref[...]
        ld = load_ref[...]
        impn = imp / (jnp.sum(imp) + EPS)
        ldn = ld / (jnp.sum(ld) + EPS)
        d2 = (impn - ldn) ** 2
        aux_ref[...] = jnp.sum(d2, axis=1, keepdims=True) / E * ua_ref[0, 0]


def kernel(x, W, use_aux_loss):
    ua = jnp.asarray(use_aux_loss, jnp.float32).reshape(1, 1)
    gates, aux = pl.pallas_call(
        _router_kernel,
        grid=(GRID,),
        in_specs=[
            pl.BlockSpec(memory_space=pltpu.SMEM),
            pl.BlockSpec((TILE, D), lambda i: (i, 0)),
            pl.BlockSpec((E, D), lambda i: (0, 0)),
        ],
        out_specs=[
            pl.BlockSpec((TILE, E), lambda i: (i, 0)),
            pl.BlockSpec((1, 1), lambda i: (0, 0)),
        ],
        out_shape=[
            jax.ShapeDtypeStruct((N, E), jnp.float32),
            jax.ShapeDtypeStruct((1, 1), jnp.float32),
        ],
        scratch_shapes=[
            pltpu.VMEM((1, E), jnp.float32),
            pltpu.VMEM((1, E), jnp.float32),
        ],
        compiler_params=pltpu.CompilerParams(
            dimension_semantics=("arbitrary",)
        ),
    )(ua, x, W)
    return gates, aux.reshape(())
